# baseline (speedup 1.0000x reference)
import functools

import jax
import jax.numpy as jnp
from jax import lax
from jax.experimental import pallas as pl
from jax.experimental.pallas import tpu as pltpu
from jax.experimental.pallas import tpu_sc as plsc

_VOCAB = 1000000
_BATCH = 16384
_HIST = 200
_N = _BATCH * _HIST
_NW = 32
_PER_W = _N // _NW
_CHUNK = 10240
_NCHUNK = _PER_W // _CHUNK
_NBUF = 3
_STAGE = 65536
_BOUNCE = 8192
_NSTAGE = _STAGE // _BOUNCE
_TAIL0 = 15 * _STAGE
_TBOUNCE = 8480

_mesh = plsc.VectorSubcoreMesh(core_axis_name="c", subcore_axis_name="s")


@functools.partial(
    pl.kernel,
    mesh=_mesh,
    out_type=jax.ShapeDtypeStruct((_N,), jnp.int32),
    scratch_types=(
        [pltpu.VMEM_SHARED((_VOCAB,), jnp.int32)]
        + [pltpu.VMEM((_CHUNK,), jnp.int32) for _ in range(2 * _NBUF)]
        + [pltpu.SemaphoreType.DMA((_NBUF,)) for _ in range(3)]
    ),
)
def _lookup(x_hbm, table_hbm, out_hbm, table_sp, i0, i1, i2, v0, v1, v2,
            sem_i, sem_g, sem_w):
    idx_v = [i0, i1, i2]
    vals_v = [v0, v1, v2]
    s = lax.axis_index("s")
    wid = s * 2 + lax.axis_index("c")
    base = wid * _PER_W

    def stage(hops):
        bufs = [i0, v0, i1]

        def hop_load(j):
            off, size = hops[j]
            return pltpu.async_copy(
                table_hbm.at[pl.ds(off, size)],
                bufs[j % 3].at[pl.ds(0, size)], sem_i.at[j % 3])

        def hop_store(j):
            off, size = hops[j]
            return pltpu.async_copy(
                bufs[j % 3].at[pl.ds(0, size)],
                table_sp.at[pl.ds(off, size)], sem_g.at[j % 3])

        n = len(hops)
        hl = {0: hop_load(0)}
        hs = {}
        for j in range(n):
            hl[j].wait()
            if j >= 2:
                hs[j - 2].wait()
            if j + 1 < n:
                hl[j + 1] = hop_load(j + 1)
            hs[j] = hop_store(j)
        for j in range(max(0, n - 2), n):
            hs[j].wait()

    @pl.when(s < 15)
    def _():
        stage([(s * _STAGE + j * _BOUNCE, _BOUNCE) for j in range(_NSTAGE)])

    @pl.when(s == 15)
    def _():
        stage([(_TAIL0 + j * _TBOUNCE, _TBOUNCE) for j in range(2)])

    plsc.subcore_barrier()

    def idx_load(g):
        b = g % _NBUF
        return pltpu.async_copy(
            x_hbm.at[pl.ds(base + g * _CHUNK, _CHUNK)], idx_v[b], sem_i.at[b])

    def gather_sp(g):
        b = g % _NBUF
        return pltpu.async_copy(table_sp.at[idx_v[b]], vals_v[b],
                                sem_g.at[b])

    def writeback(g):
        b = g % _NBUF
        return pltpu.async_copy(
            vals_v[b], out_hbm.at[pl.ds(base + g * _CHUNK, _CHUNK)],
            sem_w.at[b])

    h_i = {}
    h_g = {}
    h_w = {}
    for g in range(_NBUF):
        h_i[g] = idx_load(g)
    for g in range(_NCHUNK):
        h_i[g].wait()
        if g >= _NBUF:
            h_w[g - _NBUF].wait()
        h_g[g] = gather_sp(g)
        if g >= 1:
            h_g[g - 1].wait()
            h_w[g - 1] = writeback(g - 1)
            if g + _NBUF - 1 < _NCHUNK:
                h_i[g + _NBUF - 1] = idx_load(g + _NBUF - 1)
    h_g[_NCHUNK - 1].wait()
    h_w[_NCHUNK - 1] = writeback(_NCHUNK - 1)
    for g in range(_NCHUNK - _NBUF, _NCHUNK):
        h_w[g].wait()


def kernel(x, table):
    out = _lookup(x.reshape(_N), table)
    return out.reshape(x.shape)


# speedup vs baseline: 1.0499x; 1.0499x over previous
import functools

import jax
import jax.numpy as jnp
from jax import lax
from jax.experimental import pallas as pl
from jax.experimental.pallas import tpu as pltpu
from jax.experimental.pallas import tpu_sc as plsc

_VOCAB = 1000000
_BATCH = 16384
_HIST = 200
_N = _BATCH * _HIST
_NW = 32
_PER_W = _N // _NW
_CHUNK = 10240
_NCHUNK = _PER_W // _CHUNK
_NBUF = 3

_scalar_mesh = plsc.ScalarSubcoreMesh(axis_name="c")
_vector_mesh = plsc.VectorSubcoreMesh(core_axis_name="c", subcore_axis_name="s")


def _scs_body(x_hbm, table_hbm, out_hbm, table_sp, i0, i1, i2, v0, v1, v2,
              sem_i, sem_g, sem_w, sem_stage):
    del x_hbm, out_hbm, i0, i1, i2, v0, v1, v2, sem_i, sem_g, sem_w
    c = lax.axis_index("c")
    pltpu.sync_copy(table_hbm, table_sp)
    for j in range(16):
        pl.semaphore_signal(sem_stage, 1, device_id={"c": c, "s": j})


def _tec_body(x_hbm, table_hbm, out_hbm, table_sp, i0, i1, i2, v0, v1, v2,
              sem_i, sem_g, sem_w, sem_stage):
    del table_hbm
    idx_v = [i0, i1, i2]
    vals_v = [v0, v1, v2]
    s = lax.axis_index("s")
    wid = s * 2 + lax.axis_index("c")
    base = wid * _PER_W

    def idx_load(g):
        b = g % _NBUF
        return pltpu.async_copy(
            x_hbm.at[pl.ds(base + g * _CHUNK, _CHUNK)], idx_v[b], sem_i.at[b])

    def gather(g):
        b = g % _NBUF
        return pltpu.async_copy(table_sp.at[idx_v[b]], vals_v[b],
                                sem_g.at[b])

    def writeback(g):
        b = g % _NBUF
        return pltpu.async_copy(
            vals_v[b], out_hbm.at[pl.ds(base + g * _CHUNK, _CHUNK)],
            sem_w.at[b])

    h_i = {}
    h_g = {}
    h_w = {}
    for g in range(_NBUF):
        h_i[g] = idx_load(g)
    pl.semaphore_wait(sem_stage, 1)
    for g in range(_NCHUNK):
        h_i[g].wait()
        if g >= _NBUF:
            h_w[g - _NBUF].wait()
        h_g[g] = gather(g)
        if g >= 1:
            h_g[g - 1].wait()
            h_w[g - 1] = writeback(g - 1)
            if g + _NBUF - 1 < _NCHUNK:
                h_i[g + _NBUF - 1] = idx_load(g + _NBUF - 1)
    h_g[_NCHUNK - 1].wait()
    h_w[_NCHUNK - 1] = writeback(_NCHUNK - 1)
    for g in range(_NCHUNK - _NBUF, _NCHUNK):
        h_w[g].wait()


_lookup = pl.kernel(
    [_scs_body, _tec_body],
    out_type=jax.ShapeDtypeStruct((_N,), jnp.int32),
    mesh=[_scalar_mesh, _vector_mesh],
    scratch_types=(
        [pltpu.VMEM_SHARED((_VOCAB,), jnp.int32)]
        + [(pltpu.VMEM @ _vector_mesh)((_CHUNK,), jnp.int32)
           for _ in range(2 * _NBUF)]
        + [(pltpu.SEMAPHORE @ _vector_mesh)((_NBUF,),
                                            pltpu.SemaphoreType.DMA.dtype)
           for _ in range(3)]
        + [pltpu.SemaphoreType.REGULAR @ _vector_mesh]
    ),
)


def kernel(x, table):
    out = _lookup(x.reshape(_N), table)
    return out.reshape(x.shape)
